# Initial kernel scaffold; baseline (speedup 1.0000x reference)
import jax
import jax.numpy as jnp
from jax.experimental import pallas as pl


def kernel(x_nchw, weight_oihw, bias):
    raise NotImplementedError("write your pallas kernel here")


# speedup vs baseline: 13.6919x; 13.6919x over previous
import jax
import jax.numpy as jnp
from jax.experimental import pallas as pl
from jax.experimental.pallas import tpu as pltpu


def kernel(x_nchw, weight_oihw, bias):
    N, C, H, W = x_nchw.shape
    Ho, Wo = H // 2, W // 2
    C2 = 2 * C

    xr = (
        jnp.transpose(x_nchw.reshape(N, C, Ho, 2, Wo, 2), (0, 2, 3, 4, 5, 1))
        .reshape(N, Ho, 2, Wo, C2)
        .astype(jnp.bfloat16)
    )

    wt = (
        jnp.transpose(weight_oihw, (2, 3, 1, 0))
        .reshape(9, C, C)
        .astype(jnp.bfloat16)
    )
    b2 = bias.reshape(1, C)

    def body(x_ref, w_ref, b_ref, o_ref):
        xe = x_ref[0, :, 0, :, :]
        xo = x_ref[0, :, 1, :, :]
        zrow = jnp.zeros((1, Wo, C2), jnp.bfloat16)
        xo_up = jnp.concatenate([zrow, xo[:-1]], axis=0)
        zcol = jnp.zeros((Ho, 1, C), jnp.bfloat16)

        acc = jnp.zeros((Ho * Wo, C), jnp.float32)
        for kh, src in ((0, xo_up), (1, xe), (2, xo)):
            ev = src[:, :, :C]
            od = src[:, :, C:]
            od_l = jnp.concatenate([zcol, od[:, :-1, :]], axis=1)
            for kw, tap in ((0, od_l), (1, ev), (2, od)):
                acc = acc + jnp.dot(
                    tap.reshape(Ho * Wo, C),
                    w_ref[kh * 3 + kw],
                    preferred_element_type=jnp.float32,
                )
        acc = acc + b_ref[0].astype(jnp.float32)
        o_ref[0] = acc.reshape(Ho, Wo, C)

    out = pl.pallas_call(
        body,
        out_shape=jax.ShapeDtypeStruct((N, Ho, Wo, C), jnp.float32),
        grid=(N,),
        in_specs=[
            pl.BlockSpec((1, Ho, 2, Wo, C2), lambda n: (n, 0, 0, 0, 0)),
            pl.BlockSpec((9, C, C), lambda n: (0, 0, 0)),
            pl.BlockSpec((1, C), lambda n: (0, 0)),
        ],
        out_specs=pl.BlockSpec((1, Ho, Wo, C), lambda n: (n, 0, 0, 0)),
        compiler_params=pltpu.CompilerParams(
            dimension_semantics=("parallel",),
            vmem_limit_bytes=64 * 1024 * 1024,
        ),
    )(xr, wt, b2)

    return jnp.transpose(out, (0, 3, 1, 2))


# speedup vs baseline: 20.0755x; 1.4662x over previous
import jax
import jax.numpy as jnp
from jax.experimental import pallas as pl
from jax.experimental.pallas import tpu as pltpu


def kernel(x_nchw, weight_oihw, bias):
    N, C, H, W = x_nchw.shape
    Ho, Wo = H // 2, W // 2
    C2 = 2 * C

    xr = (
        jnp.transpose(x_nchw, (0, 2, 3, 1))
        .astype(jnp.bfloat16)
        .reshape(N, Ho, 2, Wo, C2)
    )

    wt = (
        jnp.transpose(weight_oihw, (2, 3, 0, 1))
        .reshape(9, C, C)
        .astype(jnp.bfloat16)
    )
    b2 = bias.reshape(C, 1)

    def body(x_ref, w_ref, b_ref, o_ref):
        xe = x_ref[0, :, 0, :, :]
        xo = x_ref[0, :, 1, :, :]
        zrow = jnp.zeros((1, Wo, C2), jnp.bfloat16)
        xo_up = jnp.concatenate([zrow, xo[:-1]], axis=0)
        zcol = jnp.zeros((Ho, 1, C), jnp.bfloat16)

        acc = jnp.zeros((C, Ho * Wo), jnp.float32)
        dn = (((1,), (1,)), ((), ()))
        for kh, src in ((0, xo_up), (1, xe), (2, xo)):
            ev = src[:, :, :C]
            od = src[:, :, C:]
            od_l = jnp.concatenate([zcol, od[:, :-1, :]], axis=1)
            for kw, tap in ((0, od_l), (1, ev), (2, od)):
                acc = acc + jax.lax.dot_general(
                    w_ref[kh * 3 + kw],
                    tap.reshape(Ho * Wo, C),
                    dimension_numbers=dn,
                    preferred_element_type=jnp.float32,
                )
        o_ref[0] = acc + b_ref[:, :]

    out = pl.pallas_call(
        body,
        out_shape=jax.ShapeDtypeStruct((N, C, Ho * Wo), jnp.float32),
        grid=(N,),
        in_specs=[
            pl.BlockSpec((1, Ho, 2, Wo, C2), lambda n: (n, 0, 0, 0, 0)),
            pl.BlockSpec((9, C, C), lambda n: (0, 0, 0)),
            pl.BlockSpec((C, 1), lambda n: (0, 0)),
        ],
        out_specs=pl.BlockSpec((1, C, Ho * Wo), lambda n: (n, 0, 0)),
        compiler_params=pltpu.CompilerParams(
            dimension_semantics=("parallel",),
            vmem_limit_bytes=64 * 1024 * 1024,
        ),
    )(xr, wt, b2)

    return out.reshape(N, C, Ho, Wo)


# speedup vs baseline: 20.1877x; 1.0056x over previous
import jax
import jax.numpy as jnp
from jax.experimental import pallas as pl
from jax.experimental.pallas import tpu as pltpu


def kernel(x_nchw, weight_oihw, bias):
    N, C, H, W = x_nchw.shape
    Ho, Wo = H // 2, W // 2
    HW = H * W
    M = Ho * Wo

    x2 = x_nchw.reshape(N, C, HW)

    wt = (
        jnp.transpose(weight_oihw, (2, 3, 0, 1))
        .reshape(9, C, C)
        .astype(jnp.bfloat16)
    )
    b2 = bias.reshape(C, 1)
    ident = jnp.eye(C, dtype=jnp.bfloat16)

    def body(x_ref, w_ref, b_ref, i_ref, o_ref, sa_ref, sb_ref):
        xb = x_ref[0].astype(jnp.bfloat16)
        xt = jax.lax.dot_general(
            xb,
            i_ref[:, :],
            dimension_numbers=(((0,), (0,)), ((), ())),
            preferred_element_type=jnp.float32,
        )
        sa_ref[:, :, :] = xt[:, :128].reshape(H, W, 128)
        sb_ref[:, :, :] = xt[:, 128:].reshape(H, W, 128)

        def phase(rp, cp):
            lo = sa_ref[pl.ds(rp, Ho, 2), pl.ds(cp, Wo, 2), :]
            hi = sb_ref[pl.ds(rp, Ho, 2), pl.ds(cp, Wo, 2), :]
            return jnp.concatenate([lo, hi], axis=-1).astype(jnp.bfloat16)

        t11 = phase(0, 0)
        t12 = phase(0, 1)
        t21 = phase(1, 0)
        t22 = phase(1, 1)

        zrow = jnp.zeros((1, Wo, C), jnp.bfloat16)
        zcol = jnp.zeros((Ho, 1, C), jnp.bfloat16)

        def rshift(t):
            return jnp.concatenate([zrow, t[:-1]], axis=0)

        def cshift(t):
            return jnp.concatenate([zcol, t[:, :-1, :]], axis=1)

        taps = (
            (0, rshift(cshift(t22))), (1, rshift(t21)), (2, rshift(t22)),
            (3, cshift(t12)), (4, t11), (5, t12),
            (6, cshift(t22)), (7, t21), (8, t22),
        )
        dn = (((1,), (1,)), ((), ()))
        acc = jnp.zeros((C, M), jnp.float32)
        for t, tap in taps:
            acc = acc + jax.lax.dot_general(
                w_ref[t],
                tap.reshape(M, C),
                dimension_numbers=dn,
                preferred_element_type=jnp.float32,
            )
        o_ref[0] = acc + b_ref[:, :]

    out = pl.pallas_call(
        body,
        out_shape=jax.ShapeDtypeStruct((N, C, M), jnp.float32),
        grid=(N,),
        in_specs=[
            pl.BlockSpec((1, C, HW), lambda n: (n, 0, 0)),
            pl.BlockSpec((9, C, C), lambda n: (0, 0, 0)),
            pl.BlockSpec((C, 1), lambda n: (0, 0)),
            pl.BlockSpec((C, C), lambda n: (0, 0)),
        ],
        out_specs=pl.BlockSpec((1, C, M), lambda n: (n, 0, 0)),
        scratch_shapes=[
            pltpu.VMEM((H, W, 128), jnp.float32),
            pltpu.VMEM((H, W, 128), jnp.float32),
        ],
        compiler_params=pltpu.CompilerParams(
            dimension_semantics=("parallel",),
            vmem_limit_bytes=96 * 1024 * 1024,
        ),
    )(x2, wt, b2, ident)

    return out.reshape(N, C, Ho, Wo)
